# baseline (speedup 1.0000x reference)
import functools

import jax
import jax.numpy as jnp
from jax import lax
from jax.experimental import pallas as pl
from jax.experimental.pallas import tpu as pltpu
from jax.experimental.pallas import tpu_sc as plsc

N_ROWS = 1_000_000
D = 32
SCALE = D ** 0.5

NC = 2
NS = 16
NW = NC * NS

B_TOTAL = 16384 * 50
B_PER_W = B_TOTAL // NW
CHUNK = 128
GROUP = 8
ROWS_PER_GROUP = CHUNK * GROUP
GROUPS = B_PER_W // ROWS_PER_GROUP
CHUNKS_PER_W = B_PER_W // CHUNK


_SCALE_BLOCK_ROWS = 8000


def _scale_body(x_ref, o_ref):
    o_ref[...] = x_ref[...] * SCALE


def _scale_table(weight):
    return pl.pallas_call(
        _scale_body,
        out_shape=jax.ShapeDtypeStruct((N_ROWS, D), jnp.float32),
        grid=(N_ROWS // _SCALE_BLOCK_ROWS,),
        in_specs=[pl.BlockSpec((_SCALE_BLOCK_ROWS, D), lambda i: (i, 0))],
        out_specs=pl.BlockSpec((_SCALE_BLOCK_ROWS, D), lambda i: (i, 0)),
    )(weight)


def _gather_body(idx_hbm, table_hbm, out_hbm, idx_v, rows_v, gsem):
    c = lax.axis_index("c")
    s = lax.axis_index("s")
    wid = s * NC + c
    pltpu.sync_copy(idx_hbm.at[wid], idx_v)
    base = wid * B_PER_W

    @pl.loop(0, GROUPS)
    def _group(g):
        descs = []
        for b in range(GROUP):
            d = pltpu.async_copy(
                table_hbm.at[idx_v.at[g * GROUP + b]],
                rows_v.at[pl.ds(b * CHUNK, CHUNK)],
                gsem,
            )
            descs.append(d)
        for d in descs:
            d.wait()
        pltpu.sync_copy(rows_v, out_hbm.at[pl.ds(base + g * ROWS_PER_GROUP,
                                                 ROWS_PER_GROUP)])


def _gather_rows(idx_grouped, table):
    mesh = plsc.VectorSubcoreMesh(core_axis_name="c", subcore_axis_name="s")
    k = pl.kernel(
        _gather_body,
        out_type=jax.ShapeDtypeStruct((B_TOTAL, D), jnp.float32),
        mesh=mesh,
        compiler_params=pltpu.CompilerParams(use_tc_tiling_on_sc=False),
        scratch_types=[
            pltpu.VMEM((CHUNKS_PER_W, CHUNK), jnp.int32),
            pltpu.VMEM((ROWS_PER_GROUP, D), jnp.float32),
            pltpu.SemaphoreType.DMA,
        ],
    )
    return k(idx_grouped, table)


def kernel(indices, weight):
    masked = _scale_table(weight)
    idx_grouped = indices.reshape(NW, CHUNKS_PER_W, CHUNK).astype(jnp.int32)
    lu = _gather_rows(idx_grouped, masked)
    return lu.reshape(indices.shape + (D,)), masked


# speedup vs baseline: 1.1683x; 1.1683x over previous
import functools

import jax
import jax.numpy as jnp
from jax import lax
from jax.experimental import pallas as pl
from jax.experimental.pallas import tpu as pltpu
from jax.experimental.pallas import tpu_sc as plsc

N_ROWS = 1_000_000
D = 32
SCALE = D ** 0.5

NC = 2
NS = 16
NW = NC * NS

B_TOTAL = 16384 * 50
B_PER_W = B_TOTAL // NW
CHUNK = 128
GROUP = 8
ROWS_PER_GROUP = CHUNK * GROUP
GROUPS = B_PER_W // ROWS_PER_GROUP
CHUNKS_PER_W = B_PER_W // CHUNK


_W128_ROWS = N_ROWS * D // 128
_SCALE_BLOCK_ROWS = 2000


def _scale_body(x_ref, o_ref):
    o_ref[...] = x_ref[...] * SCALE


def _scale_table(weight):
    w128 = weight.reshape(_W128_ROWS, 128)
    m128 = pl.pallas_call(
        _scale_body,
        out_shape=jax.ShapeDtypeStruct((_W128_ROWS, 128), jnp.float32),
        grid=(_W128_ROWS // _SCALE_BLOCK_ROWS,),
        in_specs=[pl.BlockSpec((_SCALE_BLOCK_ROWS, 128), lambda i: (i, 0))],
        out_specs=pl.BlockSpec((_SCALE_BLOCK_ROWS, 128), lambda i: (i, 0)),
    )(w128)
    return m128.reshape(N_ROWS, D)


def _gather_body(idx_hbm, table_hbm, out_hbm, idx_v, rows_v, gsem):
    c = lax.axis_index("c")
    s = lax.axis_index("s")
    wid = s * NC + c
    pltpu.sync_copy(idx_hbm.at[wid], idx_v)
    base = wid * B_PER_W

    @pl.loop(0, GROUPS)
    def _group(g):
        descs = []
        for b in range(GROUP):
            d = pltpu.async_copy(
                table_hbm.at[idx_v.at[g * GROUP + b]],
                rows_v.at[pl.ds(b * CHUNK, CHUNK)],
                gsem,
            )
            descs.append(d)
        for d in descs:
            d.wait()
        pltpu.sync_copy(rows_v, out_hbm.at[pl.ds(base + g * ROWS_PER_GROUP,
                                                 ROWS_PER_GROUP)])


def _gather_rows(idx_grouped, table):
    mesh = plsc.VectorSubcoreMesh(core_axis_name="c", subcore_axis_name="s")
    k = pl.kernel(
        _gather_body,
        out_type=jax.ShapeDtypeStruct((B_TOTAL, D), jnp.float32),
        mesh=mesh,
        compiler_params=pltpu.CompilerParams(use_tc_tiling_on_sc=False),
        scratch_types=[
            pltpu.VMEM((CHUNKS_PER_W, CHUNK), jnp.int32),
            pltpu.VMEM((ROWS_PER_GROUP, D), jnp.float32),
            pltpu.SemaphoreType.DMA,
        ],
    )
    return k(idx_grouped, table)


def kernel(indices, weight):
    masked = _scale_table(weight)
    idx_grouped = indices.reshape(NW, CHUNKS_PER_W, CHUNK).astype(jnp.int32)
    lu = _gather_rows(idx_grouped, masked)
    return lu.reshape(indices.shape + (D,)), masked


# speedup vs baseline: 1.1891x; 1.0178x over previous
import jax
import jax.numpy as jnp
from jax import lax
from jax.experimental import pallas as pl
from jax.experimental.pallas import tpu as pltpu
from jax.experimental.pallas import tpu_sc as plsc

N_ROWS = 1_000_000
D = 32
SCALE = D ** 0.5

NC = 2
NS = 16
NW = NC * NS

A_ROWS_PER_W = N_ROWS // NW
A_CHUNK = 625
A_CHUNKS = A_ROWS_PER_W // A_CHUNK

B_TOTAL = 16384 * 50
B_PER_W = B_TOTAL // NW
CHUNK = 128
GROUP = 4
ROWS_PER_GROUP = CHUNK * GROUP
GROUPS = B_PER_W // ROWS_PER_GROUP
CHUNKS_PER_W = B_PER_W // CHUNK

MUL_UNROLL = 8


def _scale_buf(buf, n_rows):
    @pl.loop(0, n_rows, unroll=MUL_UNROLL)
    def _row(r):
        buf[r, pl.ds(0, 16)] = buf[r, pl.ds(0, 16)] * SCALE
        buf[r, pl.ds(16, 16)] = buf[r, pl.ds(16, 16)] * SCALE


def _body(idx_hbm, w_hbm, lu_hbm, masked_hbm,
          idx_v, a0, a1, r0, r1,
          a_in0, a_in1, a_out0, a_out1,
          g_in0, g_in1, g_out0, g_out1):
    c = lax.axis_index("c")
    s = lax.axis_index("s")
    wid = s * NC + c

    a_base = wid * A_ROWS_PER_W
    abufs = (a0, a1)
    a_isems = (a_in0, a_in1)
    a_osems = (a_out0, a_out1)

    def a_src(ci):
        return w_hbm.at[pl.ds(a_base + ci * A_CHUNK, A_CHUNK)]

    def a_dst(ci):
        return masked_hbm.at[pl.ds(a_base + ci * A_CHUNK, A_CHUNK)]

    pltpu.async_copy(a_src(0), abufs[0], a_isems[0])

    @pl.loop(0, A_CHUNKS // 2)
    def _a(p):
        for b in range(2):
            ci = 2 * p + b
            nb = 1 - b
            pltpu.make_async_copy(a_src(ci), abufs[b], a_isems[b]).wait()
            @pl.when(ci >= 1)
            def _():
                pltpu.make_async_copy(abufs[nb], a_dst(ci - 1),
                                      a_osems[nb]).wait()
            @pl.when(ci + 1 < A_CHUNKS)
            def _():
                pltpu.async_copy(a_src(ci + 1), abufs[nb], a_isems[nb])
            _scale_buf(abufs[b], A_CHUNK)
            pltpu.async_copy(abufs[b], a_dst(ci), a_osems[b])

    pltpu.make_async_copy(abufs[1], a_dst(A_CHUNKS - 1), a_osems[1]).wait()

    pltpu.sync_copy(idx_hbm.at[wid], idx_v)
    b_base = wid * B_PER_W
    rbufs = (r0, r1)
    g_isems = (g_in0, g_in1)
    g_osems = (g_out0, g_out1)

    def fire_group(g, b):
        for j in range(GROUP):
            pltpu.async_copy(
                w_hbm.at[idx_v.at[g * GROUP + j]],
                rbufs[b].at[pl.ds(j * CHUNK, CHUNK)],
                g_isems[b],
            )

    def wait_group(g, b):
        for j in range(GROUP):
            pltpu.make_async_copy(
                w_hbm.at[idx_v.at[g * GROUP + j]],
                rbufs[b].at[pl.ds(j * CHUNK, CHUNK)],
                g_isems[b],
            ).wait()

    def out_dst(g):
        return lu_hbm.at[pl.ds(b_base + g * ROWS_PER_GROUP, ROWS_PER_GROUP)]

    fire_group(0, 0)

    @pl.loop(0, GROUPS // 2)
    def _g(p):
        for b in range(2):
            g = 2 * p + b
            nb = 1 - b
            wait_group(g, b)
            @pl.when(g >= 1)
            def _():
                pltpu.make_async_copy(rbufs[nb], out_dst(g - 1),
                                      g_osems[nb]).wait()
            @pl.when(g + 1 < GROUPS)
            def _():
                fire_group(g + 1, nb)
            _scale_buf(rbufs[b], ROWS_PER_GROUP)
            pltpu.async_copy(rbufs[b], out_dst(g), g_osems[b])

    pltpu.make_async_copy(rbufs[1], out_dst(GROUPS - 1), g_osems[1]).wait()


def _sc_fused(idx_grouped, weight):
    mesh = plsc.VectorSubcoreMesh(core_axis_name="c", subcore_axis_name="s")
    k = pl.kernel(
        _body,
        out_type=(
            jax.ShapeDtypeStruct((B_TOTAL, D), jnp.float32),
            jax.ShapeDtypeStruct((N_ROWS, D), jnp.float32),
        ),
        mesh=mesh,
        compiler_params=pltpu.CompilerParams(use_tc_tiling_on_sc=False),
        scratch_types=[
            pltpu.VMEM((CHUNKS_PER_W, CHUNK), jnp.int32),
            pltpu.VMEM((A_CHUNK, D), jnp.float32),
            pltpu.VMEM((A_CHUNK, D), jnp.float32),
            pltpu.VMEM((ROWS_PER_GROUP, D), jnp.float32),
            pltpu.VMEM((ROWS_PER_GROUP, D), jnp.float32),
        ] + [pltpu.SemaphoreType.DMA] * 8,
    )
    return k(idx_grouped, weight)


def kernel(indices, weight):
    idx_grouped = indices.reshape(NW, CHUNKS_PER_W, CHUNK).astype(jnp.int32)
    lu, masked = _sc_fused(idx_grouped, weight)
    return lu.reshape(indices.shape + (D,)), masked


# speedup vs baseline: 1.3727x; 1.1544x over previous
import jax
import jax.numpy as jnp
from jax import lax
from jax.experimental import pallas as pl
from jax.experimental.pallas import tpu as pltpu
from jax.experimental.pallas import tpu_sc as plsc

N_ROWS = 1_000_000
D = 32
SCALE = D ** 0.5

NC = 2
NS = 16
NW = NC * NS

B_TOTAL = 16384 * 50
B_PER_W = B_TOTAL // NW
CHUNK = 128
GROUP = 4
ROWS_PER_GROUP = CHUNK * GROUP
GROUPS = B_PER_W // ROWS_PER_GROUP
CHUNKS_PER_W = B_PER_W // CHUNK

MUL_UNROLL = 8


_T_BLK = 32000


def _scale_body(x_ref, o_ref):
    o_ref[...] = x_ref[...] * SCALE


def _scale_table(weight):
    wt = weight.T
    grid = (N_ROWS + _T_BLK - 1) // _T_BLK
    mt = pl.pallas_call(
        _scale_body,
        out_shape=jax.ShapeDtypeStruct((D, N_ROWS), jnp.float32),
        grid=(grid,),
        in_specs=[pl.BlockSpec((D, _T_BLK), lambda i: (0, i))],
        out_specs=pl.BlockSpec((D, _T_BLK), lambda i: (0, i)),
    )(wt)
    return mt.T


def _scale_buf(buf, n_rows):
    @pl.loop(0, n_rows, unroll=MUL_UNROLL)
    def _row(r):
        buf[r, pl.ds(0, 16)] = buf[r, pl.ds(0, 16)] * SCALE
        buf[r, pl.ds(16, 16)] = buf[r, pl.ds(16, 16)] * SCALE


def _gather_body(idx_hbm, w_hbm, lu_hbm,
                 idx_v, r0, r1,
                 g_in0, g_in1, g_out0, g_out1):
    c = lax.axis_index("c")
    s = lax.axis_index("s")
    wid = s * NC + c

    pltpu.sync_copy(idx_hbm.at[wid], idx_v)
    b_base = wid * B_PER_W
    rbufs = (r0, r1)
    g_isems = (g_in0, g_in1)
    g_osems = (g_out0, g_out1)

    def fire_group(g, b):
        for j in range(GROUP):
            pltpu.async_copy(
                w_hbm.at[idx_v.at[g * GROUP + j]],
                rbufs[b].at[pl.ds(j * CHUNK, CHUNK)],
                g_isems[b],
            )

    def wait_group(g, b):
        for j in range(GROUP):
            pltpu.make_async_copy(
                w_hbm.at[idx_v.at[g * GROUP + j]],
                rbufs[b].at[pl.ds(j * CHUNK, CHUNK)],
                g_isems[b],
            ).wait()

    def out_dst(g):
        return lu_hbm.at[pl.ds(b_base + g * ROWS_PER_GROUP, ROWS_PER_GROUP)]

    fire_group(0, 0)

    @pl.loop(0, GROUPS // 2)
    def _g(p):
        for b in range(2):
            g = 2 * p + b
            nb = 1 - b
            wait_group(g, b)
            @pl.when(g >= 1)
            def _():
                pltpu.make_async_copy(rbufs[nb], out_dst(g - 1),
                                      g_osems[nb]).wait()
            @pl.when(g + 1 < GROUPS)
            def _():
                fire_group(g + 1, nb)
            _scale_buf(rbufs[b], ROWS_PER_GROUP)
            pltpu.async_copy(rbufs[b], out_dst(g), g_osems[b])

    pltpu.make_async_copy(rbufs[1], out_dst(GROUPS - 1), g_osems[1]).wait()


def _sc_gather(idx_grouped, weight):
    mesh = plsc.VectorSubcoreMesh(core_axis_name="c", subcore_axis_name="s")
    k = pl.kernel(
        _gather_body,
        out_type=jax.ShapeDtypeStruct((B_TOTAL, D), jnp.float32),
        mesh=mesh,
        compiler_params=pltpu.CompilerParams(use_tc_tiling_on_sc=False),
        scratch_types=[
            pltpu.VMEM((CHUNKS_PER_W, CHUNK), jnp.int32),
            pltpu.VMEM((ROWS_PER_GROUP, D), jnp.float32),
            pltpu.VMEM((ROWS_PER_GROUP, D), jnp.float32),
        ] + [pltpu.SemaphoreType.DMA] * 4,
    )
    return k(idx_grouped, weight)


def kernel(indices, weight):
    masked = _scale_table(weight)
    idx_grouped = indices.reshape(NW, CHUNKS_PER_W, CHUNK).astype(jnp.int32)
    lu = _sc_gather(idx_grouped, weight)
    return lu.reshape(indices.shape + (D,)), masked


# speedup vs baseline: 1.8793x; 1.3691x over previous
import jax
import jax.numpy as jnp
from jax import lax
from jax.experimental import pallas as pl
from jax.experimental.pallas import tpu as pltpu
from jax.experimental.pallas import tpu_sc as plsc

N_ROWS = 1_000_000
D = 32
SCALE = D ** 0.5

NC = 2
NS = 16
NW = NC * NS

B_SAMPLES = 16384
SEQ = 50
S_PER_W = B_SAMPLES // NW
IDX_PER_W = S_PER_W * SEQ
GROUP = 16
ROWS_PER_GROUP = GROUP * SEQ
GROUPS = S_PER_W // GROUP
DESCS = ((0, 128), (128, 128), (256, 128), (384, 128),
         (512, 128), (640, 128), (768, 32))

MUL_UNROLL = 10


_T_BLK = 32000


def _scale_body(x_ref, o_ref):
    o_ref[...] = x_ref[...] * SCALE


def _scale_table(weight):
    wt = weight.T
    grid = (N_ROWS + _T_BLK - 1) // _T_BLK
    mt = pl.pallas_call(
        _scale_body,
        out_shape=jax.ShapeDtypeStruct((D, N_ROWS), jnp.float32),
        grid=(grid,),
        in_specs=[pl.BlockSpec((D, _T_BLK), lambda i: (0, i))],
        out_specs=pl.BlockSpec((D, _T_BLK), lambda i: (0, i)),
    )(wt)
    return mt.T


def _scale_relayout(rbuf, obuf):
    @pl.loop(0, GROUP)
    def _s(si):
        @pl.loop(0, SEQ, unroll=MUL_UNROLL)
        def _row(r):
            flat = si * SEQ + r
            obuf[si, r, pl.ds(0, 16)] = rbuf[flat, pl.ds(0, 16)] * SCALE
            obuf[si, r, pl.ds(16, 16)] = rbuf[flat, pl.ds(16, 16)] * SCALE


def _gather_body(idx_hbm, w_hbm, lu_hbm,
                 idx_v, r0, r1, obuf,
                 g_in0, g_in1, g_osem):
    c = lax.axis_index("c")
    s = lax.axis_index("s")
    wid = s * NC + c

    pltpu.sync_copy(idx_hbm.at[wid], idx_v)
    s_base = wid * S_PER_W
    rbufs = (r0, r1)
    g_isems = (g_in0, g_in1)

    def fire_group(g, b):
        for off, n in DESCS:
            pltpu.async_copy(
                w_hbm.at[idx_v.at[pl.ds(g * ROWS_PER_GROUP + off, n)]],
                rbufs[b].at[pl.ds(off, n)],
                g_isems[b],
            )

    def wait_group(g, b):
        for off, n in DESCS:
            pltpu.make_async_copy(
                w_hbm.at[idx_v.at[pl.ds(g * ROWS_PER_GROUP + off, n)]],
                rbufs[b].at[pl.ds(off, n)],
                g_isems[b],
            ).wait()

    def out_dst(g):
        return lu_hbm.at[pl.ds(s_base + g * GROUP, GROUP)]

    fire_group(0, 0)

    @pl.loop(0, GROUPS // 2)
    def _g(p):
        for b in range(2):
            g = 2 * p + b
            nb = 1 - b
            wait_group(g, b)
            @pl.when(g + 1 < GROUPS)
            def _():
                fire_group(g + 1, nb)
            @pl.when(g >= 1)
            def _():
                pltpu.make_async_copy(obuf, out_dst(g - 1), g_osem).wait()
            _scale_relayout(rbufs[b], obuf)
            pltpu.async_copy(obuf, out_dst(g), g_osem)

    pltpu.make_async_copy(obuf, out_dst(GROUPS - 1), g_osem).wait()


def _sc_gather(idx_flat, weight):
    mesh = plsc.VectorSubcoreMesh(core_axis_name="c", subcore_axis_name="s")
    k = pl.kernel(
        _gather_body,
        out_type=jax.ShapeDtypeStruct((B_SAMPLES, SEQ, D), jnp.float32),
        mesh=mesh,
        compiler_params=pltpu.CompilerParams(use_tc_tiling_on_sc=False),
        scratch_types=[
            pltpu.VMEM((IDX_PER_W,), jnp.int32),
            pltpu.VMEM((ROWS_PER_GROUP, D), jnp.float32),
            pltpu.VMEM((ROWS_PER_GROUP, D), jnp.float32),
            pltpu.VMEM((GROUP, SEQ, D), jnp.float32),
        ] + [pltpu.SemaphoreType.DMA] * 3,
    )
    return k(idx_flat, weight)


def kernel(indices, weight):
    masked = _scale_table(weight)
    idx_flat = indices.astype(jnp.int32).reshape(NW, IDX_PER_W)
    lu = _sc_gather(idx_flat, weight)
    return lu, masked


# speedup vs baseline: 1.8936x; 1.0076x over previous
import jax
import jax.numpy as jnp
from jax import lax
from jax.experimental import pallas as pl
from jax.experimental.pallas import tpu as pltpu
from jax.experimental.pallas import tpu_sc as plsc

N_ROWS = 1_000_000
D = 32
SCALE = D ** 0.5

NC = 2
NS = 16
NW = NC * NS

B_SAMPLES = 16384
SEQ = 50
S_PER_W = B_SAMPLES // NW
IDX_PER_W = S_PER_W * SEQ
GROUP = 16
ROWS_PER_GROUP = GROUP * SEQ
GROUPS = S_PER_W // GROUP
DESCS = ((0, 128), (128, 128), (256, 128), (384, 128),
         (512, 128), (640, 128), (768, 32))

MUL_UNROLL = 10


_T_BLK = 32000


def _scale_body(x_ref, o_ref):
    o_ref[...] = x_ref[...] * SCALE


def _scale_table(weight):
    wt = weight.T
    grid = (N_ROWS + _T_BLK - 1) // _T_BLK
    mt = pl.pallas_call(
        _scale_body,
        out_shape=jax.ShapeDtypeStruct((D, N_ROWS), jnp.float32),
        grid=(grid,),
        in_specs=[pl.BlockSpec((D, _T_BLK), lambda i: (0, i))],
        out_specs=pl.BlockSpec((D, _T_BLK), lambda i: (0, i)),
    )(wt)
    return mt.T


def _scale_relayout(ra, rb, obuf):
    @pl.loop(0, GROUP)
    def _s(si):
        @pl.loop(0, SEQ, unroll=MUL_UNROLL)
        def _row(r):
            flat = si * SEQ + r
            obuf[si, r, pl.ds(0, 16)] = ra[flat, :] * SCALE
            obuf[si, r, pl.ds(16, 16)] = rb[flat, :] * SCALE


def _gather_body(idx_hbm, w_hbm, lu_hbm,
                 idx_v, a0, b0, a1, b1, la0, lb0, la1, lb1, obuf,
                 g_in0, g_in1, g_osem):
    c = lax.axis_index("c")
    s = lax.axis_index("s")
    wid = s * NC + c

    pltpu.sync_copy(idx_hbm.at[wid], idx_v)
    s_base = wid * S_PER_W
    rabufs = ((a0, b0), (a1, b1))
    lists = ((la0, lb0), (la1, lb1))
    g_isems = (g_in0, g_in1)

    def build_lists(g, b):
        la, lb = lists[b]
        @pl.loop(0, ROWS_PER_GROUP // 16, unroll=5)
        def _k(k):
            v = idx_v[pl.ds(g * ROWS_PER_GROUP + k * 16, 16)] * 8
            la[pl.ds(k * 16, 16)] = v
            lb[pl.ds(k * 16, 16)] = v + 1

    def fire_group(g, b):
        ra, rb = rabufs[b]
        la, lb = lists[b]
        for off, n in DESCS:
            pltpu.async_copy(w_hbm.at[la.at[pl.ds(off, n)]],
                             ra.at[pl.ds(off, n)], g_isems[b])
            pltpu.async_copy(w_hbm.at[lb.at[pl.ds(off, n)]],
                             rb.at[pl.ds(off, n)], g_isems[b])

    def wait_group(g, b):
        ra, rb = rabufs[b]
        la, lb = lists[b]
        for off, n in DESCS:
            pltpu.make_async_copy(w_hbm.at[la.at[pl.ds(off, n)]],
                                  ra.at[pl.ds(off, n)], g_isems[b]).wait()
            pltpu.make_async_copy(w_hbm.at[lb.at[pl.ds(off, n)]],
                                  rb.at[pl.ds(off, n)], g_isems[b]).wait()

    def out_dst(g):
        return lu_hbm.at[pl.ds(s_base + g * GROUP, GROUP)]

    build_lists(0, 0)
    fire_group(0, 0)

    @pl.loop(0, GROUPS // 2)
    def _g(p):
        for b in range(2):
            g = 2 * p + b
            nb = 1 - b
            wait_group(g, b)
            @pl.when(g + 1 < GROUPS)
            def _():
                build_lists(g + 1, nb)
                fire_group(g + 1, nb)
            @pl.when(g >= 1)
            def _():
                pltpu.make_async_copy(obuf, out_dst(g - 1), g_osem).wait()
            _scale_relayout(rabufs[b][0], rabufs[b][1], obuf)
            pltpu.async_copy(obuf, out_dst(g), g_osem)

    pltpu.make_async_copy(obuf, out_dst(GROUPS - 1), g_osem).wait()


def _sc_gather(idx_flat, w16):
    mesh = plsc.VectorSubcoreMesh(core_axis_name="c", subcore_axis_name="s")
    k = pl.kernel(
        _gather_body,
        out_type=jax.ShapeDtypeStruct((B_SAMPLES, SEQ, D), jnp.float32),
        mesh=mesh,
        compiler_params=pltpu.CompilerParams(use_tc_tiling_on_sc=False),
        scratch_types=[
            pltpu.VMEM((IDX_PER_W,), jnp.int32),
            pltpu.VMEM((ROWS_PER_GROUP, 16), jnp.float32),
            pltpu.VMEM((ROWS_PER_GROUP, 16), jnp.float32),
            pltpu.VMEM((ROWS_PER_GROUP, 16), jnp.float32),
            pltpu.VMEM((ROWS_PER_GROUP, 16), jnp.float32),
            pltpu.VMEM((ROWS_PER_GROUP,), jnp.int32),
            pltpu.VMEM((ROWS_PER_GROUP,), jnp.int32),
            pltpu.VMEM((ROWS_PER_GROUP,), jnp.int32),
            pltpu.VMEM((ROWS_PER_GROUP,), jnp.int32),
            pltpu.VMEM((GROUP, SEQ, D), jnp.float32),
        ] + [pltpu.SemaphoreType.DMA] * 3,
    )
    return k(idx_flat, w16)


def kernel(indices, weight):
    masked = _scale_table(weight)
    wpad = jnp.pad(weight, ((0, 0), (0, 128 - D)))
    w16 = wpad.reshape(N_ROWS * 8, 16)
    idx_flat = indices.astype(jnp.int32).reshape(NW, IDX_PER_W)
    lu = _sc_gather(idx_flat, w16)
    return lu, masked


# speedup vs baseline: 2.5773x; 1.3611x over previous
import jax
import jax.numpy as jnp
from jax import lax
from jax.experimental import pallas as pl
from jax.experimental.pallas import tpu as pltpu
from jax.experimental.pallas import tpu_sc as plsc

N_ROWS = 1_000_000
D = 32
SCALE = D ** 0.5

NC = 2
NS = 16
NW = NC * NS

B_SAMPLES = 16384
SEQ = 50
S_PER_W = B_SAMPLES // NW
IDX_PER_W = S_PER_W * SEQ
GROUP = 16
ROWS_PER_GROUP = GROUP * SEQ
GROUPS = S_PER_W // GROUP
DESCS = ((0, 128), (128, 128), (256, 128), (384, 128),
         (512, 128), (640, 128), (768, 32))

MUL_UNROLL = 10


_T_BLK = 16384


def _scale_body(x_ref, o_ref, p_ref):
    x = x_ref[...] * SCALE
    o_ref[...] = x
    xt = x.T
    p_ref[...] = jnp.concatenate(
        [xt, jnp.zeros((_T_BLK, 128 - D), jnp.float32)], axis=1)


def _scale_table(weight):
    wt = weight.T
    grid = (N_ROWS + _T_BLK - 1) // _T_BLK
    mt, mpad = pl.pallas_call(
        _scale_body,
        out_shape=(
            jax.ShapeDtypeStruct((D, N_ROWS), jnp.float32),
            jax.ShapeDtypeStruct((N_ROWS, 128), jnp.float32),
        ),
        grid=(grid,),
        in_specs=[pl.BlockSpec((D, _T_BLK), lambda i: (0, i))],
        out_specs=(
            pl.BlockSpec((D, _T_BLK), lambda i: (0, i)),
            pl.BlockSpec((_T_BLK, 128), lambda i: (i, 0)),
        ),
    )(wt)
    return mt.T, mpad


def _scale_relayout(ra, rb, obuf):
    @pl.loop(0, GROUP)
    def _s(si):
        @pl.loop(0, SEQ, unroll=MUL_UNROLL)
        def _row(r):
            flat = si * SEQ + r
            obuf[si, r, pl.ds(0, 16)] = ra[flat, :]
            obuf[si, r, pl.ds(16, 16)] = rb[flat, :]


def _gather_body(idx_hbm, w_hbm, lu_hbm,
                 idx_v, a0, b0, a1, b1, la0, lb0, la1, lb1, obuf,
                 g_in0, g_in1, g_osem):
    c = lax.axis_index("c")
    s = lax.axis_index("s")
    wid = s * NC + c

    pltpu.sync_copy(idx_hbm.at[wid], idx_v)
    s_base = wid * S_PER_W
    rabufs = ((a0, b0), (a1, b1))
    lists = ((la0, lb0), (la1, lb1))
    g_isems = (g_in0, g_in1)

    def build_lists(g, b):
        la, lb = lists[b]
        @pl.loop(0, ROWS_PER_GROUP // 16, unroll=5)
        def _k(k):
            v = idx_v[pl.ds(g * ROWS_PER_GROUP + k * 16, 16)] * 8
            la[pl.ds(k * 16, 16)] = v
            lb[pl.ds(k * 16, 16)] = v + 1

    def fire_group(g, b):
        ra, rb = rabufs[b]
        la, lb = lists[b]
        for off, n in DESCS:
            pltpu.async_copy(w_hbm.at[la.at[pl.ds(off, n)]],
                             ra.at[pl.ds(off, n)], g_isems[b])
            pltpu.async_copy(w_hbm.at[lb.at[pl.ds(off, n)]],
                             rb.at[pl.ds(off, n)], g_isems[b])

    def wait_group(g, b):
        ra, rb = rabufs[b]
        la, lb = lists[b]
        for off, n in DESCS:
            pltpu.make_async_copy(w_hbm.at[la.at[pl.ds(off, n)]],
                                  ra.at[pl.ds(off, n)], g_isems[b]).wait()
            pltpu.make_async_copy(w_hbm.at[lb.at[pl.ds(off, n)]],
                                  rb.at[pl.ds(off, n)], g_isems[b]).wait()

    def out_dst(g):
        return lu_hbm.at[pl.ds(s_base + g * GROUP, GROUP)]

    build_lists(0, 0)
    fire_group(0, 0)

    @pl.loop(0, GROUPS // 2)
    def _g(p):
        for b in range(2):
            g = 2 * p + b
            nb = 1 - b
            wait_group(g, b)
            @pl.when(g + 1 < GROUPS)
            def _():
                build_lists(g + 1, nb)
                fire_group(g + 1, nb)
            @pl.when(g >= 1)
            def _():
                pltpu.make_async_copy(obuf, out_dst(g - 1), g_osem).wait()
            _scale_relayout(rabufs[b][0], rabufs[b][1], obuf)
            pltpu.async_copy(obuf, out_dst(g), g_osem)

    pltpu.make_async_copy(obuf, out_dst(GROUPS - 1), g_osem).wait()


def _sc_gather(idx_flat, w16):
    mesh = plsc.VectorSubcoreMesh(core_axis_name="c", subcore_axis_name="s")
    k = pl.kernel(
        _gather_body,
        out_type=jax.ShapeDtypeStruct((B_SAMPLES, SEQ, D), jnp.float32),
        mesh=mesh,
        compiler_params=pltpu.CompilerParams(use_tc_tiling_on_sc=False),
        scratch_types=[
            pltpu.VMEM((IDX_PER_W,), jnp.int32),
            pltpu.VMEM((ROWS_PER_GROUP, 16), jnp.float32),
            pltpu.VMEM((ROWS_PER_GROUP, 16), jnp.float32),
            pltpu.VMEM((ROWS_PER_GROUP, 16), jnp.float32),
            pltpu.VMEM((ROWS_PER_GROUP, 16), jnp.float32),
            pltpu.VMEM((ROWS_PER_GROUP,), jnp.int32),
            pltpu.VMEM((ROWS_PER_GROUP,), jnp.int32),
            pltpu.VMEM((ROWS_PER_GROUP,), jnp.int32),
            pltpu.VMEM((ROWS_PER_GROUP,), jnp.int32),
            pltpu.VMEM((GROUP, SEQ, D), jnp.float32),
        ] + [pltpu.SemaphoreType.DMA] * 3,
    )
    return k(idx_flat, w16)


def kernel(indices, weight):
    masked, mpad = _scale_table(weight)
    w16 = mpad.reshape(N_ROWS * 8, 16)
    idx_flat = indices.astype(jnp.int32).reshape(NW, IDX_PER_W)
    lu = _sc_gather(idx_flat, w16)
    return lu, masked
